# baseline (device time: 40468 ns/iter reference)
import jax
import jax.numpy as jnp
from jax import lax
from jax.experimental import pallas as pl
from jax.experimental.pallas import tpu as pltpu

N_DEV = 4
B, SQ, SKV, HQ, DH = 2, 256, 1024, 16, 64
H_LOC = HQ // N_DEV
HD_LOC = H_LOC * DH
SKV_LOC = SKV // N_DEV
D_MODEL = 512


def kernel(x, Wq, K_ext, V_ext, Wo):
    xb = x.reshape(B * SQ, D_MODEL).astype(jnp.bfloat16)
    wqb = Wq.astype(jnp.bfloat16)
    kb = K_ext.reshape(B, SKV_LOC, HQ * DH).astype(jnp.bfloat16)
    vb = V_ext.reshape(B, SKV_LOC, HQ * DH).astype(jnp.bfloat16)
    wob = Wo.astype(jnp.bfloat16)

    def body(x_ref, wq_ref, k_ref, v_ref, wo_ref, out_ref,
             k_all, v_all, out_bufs,
             ksend, krecv, vsend, vrecv, osend, orecv):
        me = lax.axis_index("i")

        barrier = pltpu.get_barrier_semaphore()
        for j in range(N_DEV):
            @pl.when(j != me)
            def _(j=j):
                pl.semaphore_signal(barrier, inc=1, device_id=(j,),
                                    device_id_type=pl.DeviceIdType.MESH)
        pl.semaphore_wait(barrier, N_DEV - 1)

        ksends, vsends = [], []
        for j in range(N_DEV):
            kd = pltpu.make_async_remote_copy(
                src_ref=k_ref.at[:, :, pl.ds(j * HD_LOC, HD_LOC)],
                dst_ref=k_all.at[me],
                send_sem=ksend.at[j], recv_sem=krecv.at[me],
                device_id=(j,), device_id_type=pl.DeviceIdType.MESH)
            vd = pltpu.make_async_remote_copy(
                src_ref=v_ref.at[:, :, pl.ds(j * HD_LOC, HD_LOC)],
                dst_ref=v_all.at[me],
                send_sem=vsend.at[j], recv_sem=vrecv.at[me],
                device_id=(j,), device_id_type=pl.DeviceIdType.MESH)
            ksends.append(kd)
            vsends.append(vd)

            @pl.when(j != me)
            def _(kd=kd, vd=vd):
                kd.start()
                vd.start()

            @pl.when(j == me)
            def _(j=j):
                k_all[j, :, :, :] = k_ref[:, :, j * HD_LOC:(j + 1) * HD_LOC]
                v_all[j, :, :, :] = v_ref[:, :, j * HD_LOC:(j + 1) * HD_LOC]

        q = jnp.dot(x_ref[...], wq_ref[...],
                    preferred_element_type=jnp.float32)
        q = (q * 0.125).astype(jnp.bfloat16)

        for j in range(N_DEV):
            krd = pltpu.make_async_remote_copy(
                src_ref=k_ref.at[:, :, pl.ds(0, HD_LOC)],
                dst_ref=k_all.at[j],
                send_sem=ksend.at[j], recv_sem=krecv.at[j],
                device_id=(j,), device_id_type=pl.DeviceIdType.MESH)
            vrd = pltpu.make_async_remote_copy(
                src_ref=v_ref.at[:, :, pl.ds(0, HD_LOC)],
                dst_ref=v_all.at[j],
                send_sem=vsend.at[j], recv_sem=vrecv.at[j],
                device_id=(j,), device_id_type=pl.DeviceIdType.MESH)

            @pl.when(j != me)
            def _(krd=krd, vrd=vrd):
                krd.wait_recv()
                vrd.wait_recv()

        row_blk = lax.broadcasted_iota(jnp.int32, (SQ, SKV), 0) // 64
        col_blk = (lax.broadcasted_iota(jnp.int32, (SQ, SKV), 1) // 64) % 4
        mask = row_blk == col_blk

        parts = []
        for b in range(B):
            ctx_h = []
            for h in range(H_LOC):
                q_bh = q[b * SQ:(b + 1) * SQ, h * DH:(h + 1) * DH]
                sc = []
                for c in range(N_DEV):
                    k_cb = k_all[c, b, :, h * DH:(h + 1) * DH]
                    sc.append(lax.dot_general(
                        q_bh, k_cb, (((1,), (1,)), ((), ())),
                        preferred_element_type=jnp.float32))
                s = jnp.concatenate(sc, axis=1)
                s = jnp.where(mask, s, jnp.float32(-1e9))
                m = jnp.max(s, axis=1, keepdims=True)
                w = jnp.exp(s - m)
                w = (w / jnp.sum(w, axis=1, keepdims=True)).astype(jnp.bfloat16)
                v_bh = jnp.concatenate(
                    [v_all[c, b, :, h * DH:(h + 1) * DH] for c in range(N_DEV)],
                    axis=0)
                ctx_h.append(jnp.dot(w, v_bh,
                                     preferred_element_type=jnp.float32))
            ctx_b = jnp.concatenate(ctx_h, axis=1).astype(jnp.bfloat16)
            parts.append(jnp.dot(ctx_b, wo_ref[...],
                                 preferred_element_type=jnp.float32))

        for j in range(N_DEV):
            @pl.when(j == me)
            def _(j=j):
                for b in range(B):
                    out_bufs[j, b, :, :] = parts[b].astype(jnp.bfloat16)

        osends = []
        for j in range(N_DEV):
            od = pltpu.make_async_remote_copy(
                src_ref=out_bufs.at[me], dst_ref=out_bufs.at[me],
                send_sem=osend.at[j], recv_sem=orecv.at[me],
                device_id=(j,), device_id_type=pl.DeviceIdType.MESH)
            osends.append(od)

            @pl.when(j != me)
            def _(od=od):
                od.start()

        for j in range(N_DEV):
            ord_ = pltpu.make_async_remote_copy(
                src_ref=out_bufs.at[j], dst_ref=out_bufs.at[j],
                send_sem=osend.at[j], recv_sem=orecv.at[j],
                device_id=(j,), device_id_type=pl.DeviceIdType.MESH)

            @pl.when(j != me)
            def _(ord_=ord_):
                ord_.wait_recv()

        acc = out_bufs[0].astype(jnp.float32)
        for j in range(1, N_DEV):
            acc = acc + out_bufs[j].astype(jnp.float32)
        out_ref[...] = acc

        for j in range(N_DEV):
            @pl.when(j != me)
            def _(kd=ksends[j], vd=vsends[j], od=osends[j]):
                kd.wait_send()
                vd.wait_send()
                od.wait_send()

    return pl.pallas_call(
        body,
        out_shape=jax.ShapeDtypeStruct((B, SQ, D_MODEL), jnp.float32),
        in_specs=[pl.BlockSpec(memory_space=pltpu.VMEM)] * 5,
        out_specs=pl.BlockSpec(memory_space=pltpu.VMEM),
        scratch_shapes=[
            pltpu.VMEM((N_DEV, B, SKV_LOC, HD_LOC), jnp.bfloat16),
            pltpu.VMEM((N_DEV, B, SKV_LOC, HD_LOC), jnp.bfloat16),
            pltpu.VMEM((N_DEV, B, SQ, D_MODEL), jnp.bfloat16),
            pltpu.SemaphoreType.DMA((N_DEV,)),
            pltpu.SemaphoreType.DMA((N_DEV,)),
            pltpu.SemaphoreType.DMA((N_DEV,)),
            pltpu.SemaphoreType.DMA((N_DEV,)),
            pltpu.SemaphoreType.DMA((N_DEV,)),
            pltpu.SemaphoreType.DMA((N_DEV,)),
        ],
        compiler_params=pltpu.CompilerParams(collective_id=0),
    )(xb, wqb, kb, vb, wob)


# device time: 39395 ns/iter; 1.0272x vs baseline; 1.0272x over previous
import jax
import jax.numpy as jnp
from jax import lax
from jax.experimental import pallas as pl
from jax.experimental.pallas import tpu as pltpu

N_DEV = 4
B, SQ, SKV, HQ, DH = 2, 256, 1024, 16, 64
H_LOC = HQ // N_DEV
HD_LOC = H_LOC * DH
SKV_LOC = SKV // N_DEV
D_MODEL = 512


def kernel(x, Wq, K_ext, V_ext, Wo):
    xb = x.reshape(B * SQ, D_MODEL).astype(jnp.bfloat16)
    wqb = Wq.astype(jnp.bfloat16)
    kb = K_ext.reshape(B, SKV_LOC, HQ * DH).astype(jnp.bfloat16)
    vb = V_ext.reshape(B, SKV_LOC, HQ * DH).astype(jnp.bfloat16)
    wob = Wo.astype(jnp.bfloat16)

    SQ_Q = SQ // N_DEV

    def body(x_ref, wq_ref, k_ref, v_ref, wo_ref, out_ref,
             k_all, v_all, my_part, rs_bufs, ag_bufs,
             ksend, krecv, vsend, vrecv, rssend, rsrecv, agsend, agrecv):
        me = lax.axis_index("i")

        barrier = pltpu.get_barrier_semaphore()
        for j in range(N_DEV):
            @pl.when(j != me)
            def _(j=j):
                pl.semaphore_signal(barrier, inc=1, device_id=(j,),
                                    device_id_type=pl.DeviceIdType.MESH)
        pl.semaphore_wait(barrier, N_DEV - 1)

        ksends, vsends = [], []
        for j in range(N_DEV):
            kd = pltpu.make_async_remote_copy(
                src_ref=k_ref.at[:, :, pl.ds(j * HD_LOC, HD_LOC)],
                dst_ref=k_all.at[me],
                send_sem=ksend.at[j], recv_sem=krecv.at[me],
                device_id=(j,), device_id_type=pl.DeviceIdType.MESH)
            vd = pltpu.make_async_remote_copy(
                src_ref=v_ref.at[:, :, pl.ds(j * HD_LOC, HD_LOC)],
                dst_ref=v_all.at[me],
                send_sem=vsend.at[j], recv_sem=vrecv.at[me],
                device_id=(j,), device_id_type=pl.DeviceIdType.MESH)
            ksends.append(kd)
            vsends.append(vd)

            @pl.when(j != me)
            def _(kd=kd, vd=vd):
                kd.start()
                vd.start()

            @pl.when(j == me)
            def _(j=j):
                k_all[j, :, :, :] = k_ref[:, :, j * HD_LOC:(j + 1) * HD_LOC]
                v_all[j, :, :, :] = v_ref[:, :, j * HD_LOC:(j + 1) * HD_LOC]

        q = jnp.dot(x_ref[...], wq_ref[...],
                    preferred_element_type=jnp.float32)
        q = (q * 0.125).astype(jnp.bfloat16)

        for j in range(N_DEV):
            krd = pltpu.make_async_remote_copy(
                src_ref=k_ref.at[:, :, pl.ds(0, HD_LOC)],
                dst_ref=k_all.at[j],
                send_sem=ksend.at[j], recv_sem=krecv.at[j],
                device_id=(j,), device_id_type=pl.DeviceIdType.MESH)
            vrd = pltpu.make_async_remote_copy(
                src_ref=v_ref.at[:, :, pl.ds(0, HD_LOC)],
                dst_ref=v_all.at[j],
                send_sem=vsend.at[j], recv_sem=vrecv.at[j],
                device_id=(j,), device_id_type=pl.DeviceIdType.MESH)

            @pl.when(j != me)
            def _(krd=krd, vrd=vrd):
                krd.wait_recv()
                vrd.wait_recv()

        row_blk = lax.broadcasted_iota(jnp.int32, (SQ, SKV), 0) // 64
        col_blk = (lax.broadcasted_iota(jnp.int32, (SQ, SKV), 1) // 64) % 4
        mask = row_blk == col_blk

        parts = []
        for b in range(B):
            ctx_h = []
            for h in range(H_LOC):
                q_bh = q[b * SQ:(b + 1) * SQ, h * DH:(h + 1) * DH]
                sc = []
                for c in range(N_DEV):
                    k_cb = k_all[c, b, :, h * DH:(h + 1) * DH]
                    sc.append(lax.dot_general(
                        q_bh, k_cb, (((1,), (1,)), ((), ())),
                        preferred_element_type=jnp.float32))
                s = jnp.concatenate(sc, axis=1)
                s = jnp.where(mask, s, jnp.float32(-1e9))
                m = jnp.max(s, axis=1, keepdims=True)
                w = jnp.exp(s - m)
                w = (w / jnp.sum(w, axis=1, keepdims=True)).astype(jnp.bfloat16)
                v_bh = jnp.concatenate(
                    [v_all[c, b, :, h * DH:(h + 1) * DH] for c in range(N_DEV)],
                    axis=0)
                ctx_h.append(jnp.dot(w, v_bh,
                                     preferred_element_type=jnp.float32))
            ctx_b = jnp.concatenate(ctx_h, axis=1).astype(jnp.bfloat16)
            parts.append(jnp.dot(ctx_b, wo_ref[...],
                                 preferred_element_type=jnp.float32))

        for b in range(B):
            my_part[b, :, :] = parts[b].astype(jnp.bfloat16)

        rssends = []
        for j in range(N_DEV):
            rd = pltpu.make_async_remote_copy(
                src_ref=my_part.at[:, pl.ds(j * SQ_Q, SQ_Q), :],
                dst_ref=rs_bufs.at[me],
                send_sem=rssend.at[j], recv_sem=rsrecv.at[me],
                device_id=(j,), device_id_type=pl.DeviceIdType.MESH)
            rssends.append(rd)

            @pl.when(j != me)
            def _(rd=rd):
                rd.start()

            @pl.when(j == me)
            def _(j=j):
                rs_bufs[j, :, :, :] = my_part[:, j * SQ_Q:(j + 1) * SQ_Q, :]

        for j in range(N_DEV):
            rrd = pltpu.make_async_remote_copy(
                src_ref=my_part.at[:, pl.ds(0, SQ_Q), :],
                dst_ref=rs_bufs.at[j],
                send_sem=rssend.at[j], recv_sem=rsrecv.at[j],
                device_id=(j,), device_id_type=pl.DeviceIdType.MESH)

            @pl.when(j != me)
            def _(rrd=rrd):
                rrd.wait_recv()

        qsum = rs_bufs[0].astype(jnp.float32)
        for c in range(1, N_DEV):
            qsum = qsum + rs_bufs[c].astype(jnp.float32)

        for j in range(N_DEV):
            @pl.when(j == me)
            def _(j=j):
                ag_bufs[j, :, :, :] = qsum.astype(jnp.bfloat16)
                out_ref[:, j * SQ_Q:(j + 1) * SQ_Q, :] = qsum

        agsends = []
        for j in range(N_DEV):
            ad = pltpu.make_async_remote_copy(
                src_ref=ag_bufs.at[me], dst_ref=ag_bufs.at[me],
                send_sem=agsend.at[j], recv_sem=agrecv.at[me],
                device_id=(j,), device_id_type=pl.DeviceIdType.MESH)
            agsends.append(ad)

            @pl.when(j != me)
            def _(ad=ad):
                ad.start()

        for j in range(N_DEV):
            ard = pltpu.make_async_remote_copy(
                src_ref=ag_bufs.at[j], dst_ref=ag_bufs.at[j],
                send_sem=agsend.at[j], recv_sem=agrecv.at[j],
                device_id=(j,), device_id_type=pl.DeviceIdType.MESH)

            @pl.when(j != me)
            def _(ard=ard, j=j):
                ard.wait_recv()
                out_ref[:, j * SQ_Q:(j + 1) * SQ_Q, :] = (
                    ag_bufs[j].astype(jnp.float32))

        for j in range(N_DEV):
            @pl.when(j != me)
            def _(kd=ksends[j], vd=vsends[j], rd=rssends[j], ad=agsends[j]):
                kd.wait_send()
                vd.wait_send()
                rd.wait_send()
                ad.wait_send()

    return pl.pallas_call(
        body,
        out_shape=jax.ShapeDtypeStruct((B, SQ, D_MODEL), jnp.float32),
        in_specs=[pl.BlockSpec(memory_space=pltpu.VMEM)] * 5,
        out_specs=pl.BlockSpec(memory_space=pltpu.VMEM),
        scratch_shapes=[
            pltpu.VMEM((N_DEV, B, SKV_LOC, HD_LOC), jnp.bfloat16),
            pltpu.VMEM((N_DEV, B, SKV_LOC, HD_LOC), jnp.bfloat16),
            pltpu.VMEM((B, SQ, D_MODEL), jnp.bfloat16),
            pltpu.VMEM((N_DEV, B, SQ // N_DEV, D_MODEL), jnp.bfloat16),
            pltpu.VMEM((N_DEV, B, SQ // N_DEV, D_MODEL), jnp.bfloat16),
            pltpu.SemaphoreType.DMA((N_DEV,)),
            pltpu.SemaphoreType.DMA((N_DEV,)),
            pltpu.SemaphoreType.DMA((N_DEV,)),
            pltpu.SemaphoreType.DMA((N_DEV,)),
            pltpu.SemaphoreType.DMA((N_DEV,)),
            pltpu.SemaphoreType.DMA((N_DEV,)),
            pltpu.SemaphoreType.DMA((N_DEV,)),
            pltpu.SemaphoreType.DMA((N_DEV,)),
        ],
        compiler_params=pltpu.CompilerParams(collective_id=0),
    )(xb, wqb, kb, vb, wob)


# device time: 30651 ns/iter; 1.3203x vs baseline; 1.2853x over previous
import os

import jax
import jax.numpy as jnp
from jax import lax
from jax.experimental import pallas as pl
from jax.experimental.pallas import tpu as pltpu

_ABLATE = int(os.environ.get("ABLATE", "0"))
_DO_P1 = _ABLATE not in (3,)
_DO_P2 = _ABLATE not in (1, 3)

N_DEV = 4
B, SQ, SKV, HQ, DH = 2, 256, 1024, 16, 64
H_LOC = HQ // N_DEV
HD_LOC = H_LOC * DH
SKV_LOC = SKV // N_DEV
D_MODEL = 512
QB = SQ // N_DEV


def kernel(x, Wq, K_ext, V_ext, Wo):
    kb = K_ext.reshape(B, SKV_LOC, HQ * DH).astype(jnp.bfloat16)
    vb = V_ext.reshape(B, SKV_LOC, HQ * DH).astype(jnp.bfloat16)

    def body(x_ref, wq_ref, k_ref, v_ref, wo_ref, out_ref, *scr):
        k_all, v_all, rs_stage, rs_bufs, ag_bufs = scr[:5]
        ksend = scr[5:9]
        krecv = scr[9:13]
        vsend = scr[13:17]
        vrecv = scr[17:21]
        rssend, rsrecv, agsend, agrecv = scr[21:25]
        me = lax.axis_index("i")

        barrier = pltpu.get_barrier_semaphore()
        for j in range(N_DEV):
            @pl.when(j != me)
            def _(j=j):
                pl.semaphore_signal(barrier, inc=1, device_id=(j,),
                                    device_id_type=pl.DeviceIdType.MESH)
        pl.semaphore_wait(barrier, N_DEV - 1)

        ksends, vsends = [], []
        for qb in range(N_DEV):
            for j in range(N_DEV):
                kd = pltpu.make_async_remote_copy(
                    src_ref=k_ref.at[:, pl.ds(qb * QB, QB),
                                     pl.ds(j * HD_LOC, HD_LOC)],
                    dst_ref=k_all.at[qb, :, pl.ds(me * QB, QB)],
                    send_sem=ksend[qb].at[j], recv_sem=krecv[qb].at[me],
                    device_id=(j,), device_id_type=pl.DeviceIdType.MESH)
                vd = pltpu.make_async_remote_copy(
                    src_ref=v_ref.at[:, pl.ds(qb * QB, QB),
                                     pl.ds(j * HD_LOC, HD_LOC)],
                    dst_ref=v_all.at[qb, :, pl.ds(me * QB, QB)],
                    send_sem=vsend[qb].at[j], recv_sem=vrecv[qb].at[me],
                    device_id=(j,), device_id_type=pl.DeviceIdType.MESH)
                ksends.append(kd)
                vsends.append(vd)

                @pl.when(j == me)
                def _(qb=qb, j=j):
                    k_all[qb, :, j * QB:(j + 1) * QB, :] = (
                        k_ref[:, qb * QB:(qb + 1) * QB,
                              j * HD_LOC:(j + 1) * HD_LOC])
                    v_all[qb, :, j * QB:(j + 1) * QB, :] = (
                        v_ref[:, qb * QB:(qb + 1) * QB,
                              j * HD_LOC:(j + 1) * HD_LOC])

                if _DO_P1:
                    @pl.when(j != me)
                    def _(kd=kd, vd=vd):
                        kd.start()
                        vd.start()

        q = jnp.dot(x_ref[...].astype(jnp.bfloat16).reshape(B * SQ, D_MODEL),
                    wq_ref[...].astype(jnp.bfloat16),
                    preferred_element_type=jnp.float32)
        q = (q * 0.125).astype(jnp.bfloat16)
        wo_b = wo_ref[...].astype(jnp.bfloat16)

        hm_row = lax.broadcasted_iota(jnp.int32, (H_LOC * QB, HD_LOC), 0) // QB
        hm_col = lax.broadcasted_iota(jnp.int32, (H_LOC * QB, HD_LOC), 1) // DH
        head_mask_f32 = jnp.where(hm_row == hm_col, 1.0, 0.0)
        head_mask = head_mask_f32.astype(jnp.bfloat16)

        rssends = []
        for qb in range(N_DEV):
            for j in range(N_DEV if _DO_P1 else 0):
                krd = pltpu.make_async_remote_copy(
                    src_ref=k_ref.at[:, pl.ds(0, QB), pl.ds(0, HD_LOC)],
                    dst_ref=k_all.at[qb, :, pl.ds(j * QB, QB)],
                    send_sem=ksend[qb].at[j], recv_sem=krecv[qb].at[j],
                    device_id=(j,), device_id_type=pl.DeviceIdType.MESH)
                vrd = pltpu.make_async_remote_copy(
                    src_ref=v_ref.at[:, pl.ds(0, QB), pl.ds(0, HD_LOC)],
                    dst_ref=v_all.at[qb, :, pl.ds(j * QB, QB)],
                    send_sem=vsend[qb].at[j], recv_sem=vrecv[qb].at[j],
                    device_id=(j,), device_id_type=pl.DeviceIdType.MESH)

                @pl.when(j != me)
                def _(krd=krd, vrd=vrd):
                    krd.wait_recv()
                    vrd.wait_recv()

            for b in range(B):
                k_blk = k_all[qb, b, :, :]
                v_blk = v_all[qb, b, :, :]
                q_blk = q[b * SQ + qb * QB:b * SQ + (qb + 1) * QB, :]
                qst = jnp.concatenate([q_blk] * H_LOC, axis=0) * head_mask
                s = lax.dot_general(
                    qst, k_blk, (((1,), (1,)), ((), ())),
                    preferred_element_type=jnp.float32)
                m = jnp.max(s, axis=1, keepdims=True)
                w = jnp.exp(s - m)
                w = (w / jnp.sum(w, axis=1, keepdims=True)).astype(jnp.bfloat16)
                cs = jnp.dot(w, v_blk,
                             preferred_element_type=jnp.float32)
                ctx_q = cs[0:QB, :] * head_mask_f32[0:QB, :]
                for h in range(1, H_LOC):
                    ctx_q = ctx_q + (cs[h * QB:(h + 1) * QB, :]
                                     * head_mask_f32[h * QB:(h + 1) * QB, :])
                part_qb = jnp.dot(ctx_q.astype(jnp.bfloat16), wo_b,
                                  preferred_element_type=jnp.float32)
                rs_stage[qb, b, :, :] = part_qb.astype(jnp.bfloat16)
                if not _DO_P2:
                    out_ref[b, qb * QB:(qb + 1) * QB, :] = (
                        part_qb.astype(jnp.bfloat16))

            if _DO_P2:
                rd = pltpu.make_async_remote_copy(
                    src_ref=rs_stage.at[qb],
                    dst_ref=rs_bufs.at[me],
                    send_sem=rssend.at[qb], recv_sem=rsrecv.at[me],
                    device_id=(qb,), device_id_type=pl.DeviceIdType.MESH)
                rssends.append(rd)

                @pl.when(qb != me)
                def _(rd=rd):
                    rd.start()

                @pl.when(qb == me)
                def _(qb=qb):
                    rs_bufs[qb, :, :, :] = rs_stage[qb, :, :, :]

        do_p2 = N_DEV if _DO_P2 else 0

        for j in range(do_p2):
            rrd = pltpu.make_async_remote_copy(
                src_ref=rs_stage.at[j],
                dst_ref=rs_bufs.at[j],
                send_sem=rssend.at[j], recv_sem=rsrecv.at[j],
                device_id=(j,), device_id_type=pl.DeviceIdType.MESH)

            @pl.when(j != me)
            def _(rrd=rrd):
                rrd.wait_recv()

        if _DO_P2:
            qsum = rs_bufs[0].astype(jnp.float32)
            for c in range(1, N_DEV):
                qsum = qsum + rs_bufs[c].astype(jnp.float32)

        for j in range(do_p2):
            @pl.when(j == me)
            def _(j=j):
                qs = qsum.astype(jnp.bfloat16)
                ag_bufs[j, :, :, :] = qs
                out_ref[:, j * QB:(j + 1) * QB, :] = qs

        ag_sends = []
        for j in range(do_p2):
            ad = pltpu.make_async_remote_copy(
                src_ref=ag_bufs.at[me], dst_ref=ag_bufs.at[me],
                send_sem=agsend.at[j], recv_sem=agrecv.at[me],
                device_id=(j,), device_id_type=pl.DeviceIdType.MESH)
            ag_sends.append(ad)

            @pl.when(j != me)
            def _(ad=ad):
                ad.start()

        for j in range(do_p2):
            ard = pltpu.make_async_remote_copy(
                src_ref=ag_bufs.at[j], dst_ref=ag_bufs.at[j],
                send_sem=agsend.at[j], recv_sem=agrecv.at[j],
                device_id=(j,), device_id_type=pl.DeviceIdType.MESH)

            @pl.when(j != me)
            def _(ard=ard, j=j):
                ard.wait_recv()
                out_ref[:, j * QB:(j + 1) * QB, :] = ag_bufs[j]

        for qb in range(N_DEV if _DO_P1 else 0):
            for j in range(N_DEV):
                @pl.when(j != me)
                def _(kd=ksends[qb * N_DEV + j], vd=vsends[qb * N_DEV + j]):
                    kd.wait_send()
                    vd.wait_send()
        for j in range(do_p2):
            @pl.when(j != me)
            def _(rd=rssends[j], ad=ag_sends[j]):
                rd.wait_send()
                ad.wait_send()

    return pl.pallas_call(
        body,
        out_shape=jax.ShapeDtypeStruct((B, SQ, D_MODEL), jnp.bfloat16),
        in_specs=[pl.BlockSpec(memory_space=pltpu.VMEM)] * 5,
        out_specs=pl.BlockSpec(memory_space=pltpu.VMEM),
        scratch_shapes=[
            pltpu.VMEM((N_DEV, B, SKV_LOC, HD_LOC), jnp.bfloat16),
            pltpu.VMEM((N_DEV, B, SKV_LOC, HD_LOC), jnp.bfloat16),
            pltpu.VMEM((N_DEV, B, QB, D_MODEL), jnp.bfloat16),
            pltpu.VMEM((N_DEV, B, QB, D_MODEL), jnp.bfloat16),
            pltpu.VMEM((N_DEV, B, QB, D_MODEL), jnp.bfloat16),
            *[pltpu.SemaphoreType.DMA((N_DEV,))
              for _ in range(N_DEV)],
            *[pltpu.SemaphoreType.DMA((N_DEV,))
              for _ in range(N_DEV)],
            *[pltpu.SemaphoreType.DMA((N_DEV,))
              for _ in range(N_DEV)],
            *[pltpu.SemaphoreType.DMA((N_DEV,))
              for _ in range(N_DEV)],
            pltpu.SemaphoreType.DMA((N_DEV,)),
            pltpu.SemaphoreType.DMA((N_DEV,)),
            pltpu.SemaphoreType.DMA((N_DEV,)),
            pltpu.SemaphoreType.DMA((N_DEV,)),
        ],
        compiler_params=pltpu.CompilerParams(collective_id=0),
    )(x, Wq, kb, vb, Wo)
